# baseline (speedup 1.0000x reference)
import functools

import jax
import jax.numpy as jnp
from jax.experimental import pallas as pl
from jax.experimental.pallas import tpu as pltpu

_NSTEPS = 8


def _rk4_kernel(x_ref, a_ref, b_ref, o_ref, *, W, G, nsteps):
    CH = a_ref.shape[1]
    L = G * W

    a0 = a_ref[0]
    a1 = a_ref[1]
    a2 = a_ref[2]
    bias = jnp.broadcast_to(b_ref[...], (CH, L))

    y = jnp.concatenate(
        [x_ref[g].reshape(CH, W) for g in range(G)], axis=1)
    y = y.astype(jnp.float32)

    q = jax.lax.broadcasted_iota(jnp.int32, (1, L), 1)
    wq = q % W
    mask_l = wq != 0
    mask_r = wq != (W - 1)

    def odefunc(z):
        zl = jnp.where(mask_l, pltpu.roll(z, 1, axis=1), 0.0)
        zr = jnp.where(mask_r, pltpu.roll(z, L - 1, axis=1), 0.0)
        acc = jnp.dot(a0, zl, preferred_element_type=jnp.float32)
        acc = acc + jnp.dot(a1, z, preferred_element_type=jnp.float32)
        acc = acc + jnp.dot(a2, zr, preferred_element_type=jnp.float32)
        return jnp.tanh(acc + bias)

    dt = 1.0 / nsteps
    for _ in range(nsteps):
        k1 = odefunc(y)
        k2 = odefunc(y + (0.5 * dt) * k1)
        k3 = odefunc(y + (0.5 * dt) * k2)
        k4 = odefunc(y + dt * k3)
        y = y + (dt / 6.0) * (k1 + 2.0 * (k2 + k3) + k4)

    y = y.astype(o_ref.dtype)
    C = o_ref.shape[1]
    for g in range(G):
        o_ref[g] = y[:, g * W:(g + 1) * W].reshape(C, CH // C, W)


def kernel(x_nchw, w_oihw, b):
    N, C, H, W = x_nchw.shape
    CH = C * H

    G = max(1, 2048 // W)
    while G > 1 and (N % G != 0 or N // G < 2):
        G //= 2
    B = N // G

    bands = jnp.stack([jnp.eye(H, k=-1, dtype=w_oihw.dtype),
                       jnp.eye(H, k=0, dtype=w_oihw.dtype),
                       jnp.eye(H, k=1, dtype=w_oihw.dtype)])
    a_all = jnp.einsum('oidw,dhk->wohik', w_oihw, bands).reshape(3, CH, CH)
    b_col = jnp.repeat(b, H).reshape(CH, 1)

    fn = functools.partial(_rk4_kernel, W=W, G=G, nsteps=_NSTEPS)
    out = pl.pallas_call(
        fn,
        out_shape=jax.ShapeDtypeStruct((N, C, H, W), x_nchw.dtype),
        grid=(B,),
        in_specs=[
            pl.BlockSpec((G, C, H, W), lambda n: (n, 0, 0, 0)),
            pl.BlockSpec((3, CH, CH), lambda n: (0, 0, 0)),
            pl.BlockSpec((CH, 1), lambda n: (0, 0)),
        ],
        out_specs=pl.BlockSpec((G, C, H, W), lambda n: (n, 0, 0, 0)),
        compiler_params=pltpu.CompilerParams(
            dimension_semantics=("arbitrary",)),
    )(x_nchw, a_all, b_col)

    return out


# speedup vs baseline: 1.1792x; 1.1792x over previous
import functools

import jax
import jax.numpy as jnp
from jax.experimental import pallas as pl
from jax.experimental.pallas import tpu as pltpu

_NSTEPS = 8


def _rk4_kernel(x_ref, a_ref, b_ref, o_ref, *, W, G, nsteps):
    CH = a_ref.shape[1]
    L = G * W

    a0 = a_ref[0]
    a1 = a_ref[1]
    a2 = a_ref[2]
    bias = jnp.broadcast_to(b_ref[...], (CH, L))

    y = jnp.concatenate(
        [x_ref[g].reshape(CH, W) for g in range(G)], axis=1)
    y = y.astype(jnp.float32)

    q = jax.lax.broadcasted_iota(jnp.int32, (1, L), 1)
    wq = q % W
    mask_l = wq != 0
    mask_r = wq != (W - 1)

    def odefunc(z):
        zl = jnp.where(mask_l, pltpu.roll(z, 1, axis=1), 0.0)
        zr = jnp.where(mask_r, pltpu.roll(z, L - 1, axis=1), 0.0)
        acc = jnp.dot(a0, zl, preferred_element_type=jnp.float32)
        acc = acc + jnp.dot(a1, z, preferred_element_type=jnp.float32)
        acc = acc + jnp.dot(a2, zr, preferred_element_type=jnp.float32)
        return jnp.tanh(acc + bias)

    dt = 1.0 / nsteps

    def rk_step(_, yc):
        k1 = odefunc(yc)
        k2 = odefunc(yc + (0.5 * dt) * k1)
        k3 = odefunc(yc + (0.5 * dt) * k2)
        k4 = odefunc(yc + dt * k3)
        return yc + (dt / 6.0) * (k1 + 2.0 * (k2 + k3) + k4)

    y = jax.lax.fori_loop(0, nsteps, rk_step, y)

    y = y.astype(o_ref.dtype)
    C = o_ref.shape[1]
    for g in range(G):
        o_ref[g] = y[:, g * W:(g + 1) * W].reshape(C, CH // C, W)


def kernel(x_nchw, w_oihw, b):
    N, C, H, W = x_nchw.shape
    CH = C * H

    G = max(1, 2048 // W)
    while G > 1 and (N % G != 0 or N // G < 2):
        G //= 2
    B = N // G

    bands = jnp.stack([jnp.eye(H, k=-1, dtype=w_oihw.dtype),
                       jnp.eye(H, k=0, dtype=w_oihw.dtype),
                       jnp.eye(H, k=1, dtype=w_oihw.dtype)])
    a_all = jnp.einsum('oidw,dhk->wohik', w_oihw, bands).reshape(3, CH, CH)
    b_col = jnp.repeat(b, H).reshape(CH, 1)

    fn = functools.partial(_rk4_kernel, W=W, G=G, nsteps=_NSTEPS)
    out = pl.pallas_call(
        fn,
        out_shape=jax.ShapeDtypeStruct((N, C, H, W), x_nchw.dtype),
        grid=(B,),
        in_specs=[
            pl.BlockSpec((G, C, H, W), lambda n: (n, 0, 0, 0)),
            pl.BlockSpec((3, CH, CH), lambda n: (0, 0, 0)),
            pl.BlockSpec((CH, 1), lambda n: (0, 0)),
        ],
        out_specs=pl.BlockSpec((G, C, H, W), lambda n: (n, 0, 0, 0)),
        compiler_params=pltpu.CompilerParams(
            dimension_semantics=("arbitrary",)),
    )(x_nchw, a_all, b_col)

    return out


# speedup vs baseline: 1.3580x; 1.1517x over previous
import functools

import jax
import jax.numpy as jnp
from jax.experimental import pallas as pl
from jax.experimental.pallas import tpu as pltpu

_NSTEPS = 8


def _rk4_kernel(x_ref, a_ref, b_ref, o_ref, *, W, G, nsteps):
    CH = a_ref.shape[1]
    L = G * W

    a_cat = jnp.concatenate([a_ref[0], a_ref[1], a_ref[2]], axis=1)
    bias = jnp.broadcast_to(b_ref[...], (CH, L))

    y = jnp.concatenate(
        [x_ref[g].reshape(CH, W) for g in range(G)], axis=1)
    y = y.astype(jnp.float32)

    q = jax.lax.broadcasted_iota(jnp.int32, (1, L), 1)
    wq = q % W
    mask_l = wq != 0
    mask_r = wq != (W - 1)

    zero = jnp.bfloat16(0.0)

    def odefunc(z):
        zb = z.astype(jnp.bfloat16)
        zl = jnp.where(mask_l, pltpu.roll(zb, 1, axis=1), zero)
        zr = jnp.where(mask_r, pltpu.roll(zb, L - 1, axis=1), zero)
        zs = jnp.concatenate([zl, zb, zr], axis=0)
        acc = jnp.dot(a_cat, zs, preferred_element_type=jnp.float32)
        return jnp.tanh(acc + bias)

    dt = 1.0 / nsteps

    def rk_step(_, yc):
        k1 = odefunc(yc)
        k2 = odefunc(yc + (0.5 * dt) * k1)
        k3 = odefunc(yc + (0.5 * dt) * k2)
        k4 = odefunc(yc + dt * k3)
        return yc + (dt / 6.0) * (k1 + 2.0 * (k2 + k3) + k4)

    y = jax.lax.fori_loop(0, nsteps, rk_step, y)

    y = y.astype(o_ref.dtype)
    C = o_ref.shape[1]
    for g in range(G):
        o_ref[g] = y[:, g * W:(g + 1) * W].reshape(C, CH // C, W)


def kernel(x_nchw, w_oihw, b):
    N, C, H, W = x_nchw.shape
    CH = C * H

    G = max(1, 2048 // W)
    while G > 1 and (N % G != 0 or N // G < 2):
        G //= 2
    B = N // G

    bands = jnp.stack([jnp.eye(H, k=-1, dtype=w_oihw.dtype),
                       jnp.eye(H, k=0, dtype=w_oihw.dtype),
                       jnp.eye(H, k=1, dtype=w_oihw.dtype)])
    a_all = jnp.einsum('oidw,dhk->wohik', w_oihw, bands).reshape(3, CH, CH)
    a_all = a_all.astype(jnp.bfloat16)
    b_col = jnp.repeat(b, H).reshape(CH, 1)

    fn = functools.partial(_rk4_kernel, W=W, G=G, nsteps=_NSTEPS)
    out = pl.pallas_call(
        fn,
        out_shape=jax.ShapeDtypeStruct((N, C, H, W), x_nchw.dtype),
        grid=(B,),
        in_specs=[
            pl.BlockSpec((G, C, H, W), lambda n: (n, 0, 0, 0)),
            pl.BlockSpec((3, CH, CH), lambda n: (0, 0, 0)),
            pl.BlockSpec((CH, 1), lambda n: (0, 0)),
        ],
        out_specs=pl.BlockSpec((G, C, H, W), lambda n: (n, 0, 0, 0)),
        compiler_params=pltpu.CompilerParams(
            dimension_semantics=("arbitrary",)),
    )(x_nchw, a_all, b_col)

    return out
